# Initial kernel scaffold; baseline (speedup 1.0000x reference)
import jax
import jax.numpy as jnp
from jax.experimental import pallas as pl


def kernel(x, edge_index, W1l, W1r, b1, W2l, W2r, b2):
    raise NotImplementedError("write your pallas kernel here")


# speedup vs baseline: 10.5489x; 10.5489x over previous
import jax
import jax.numpy as jnp
from jax import lax
from jax.experimental import pallas as pl
from jax.experimental.pallas import tpu as pltpu
from jax.experimental.pallas import tpu_sc as plsc

N = 10000
E = 320000
D_IN = 128
D_HID = 64

NC, NS = 2, 16
NW = NC * NS
EPW = E // NW
CH = 80
NCHUNK = EPW // CH
NP = 10240
RP = NP // NS

_mesh = plsc.VectorSubcoreMesh(core_axis_name="c", subcore_axis_name="s")


def _proj1_body(x_ref, wl_ref, wr_ref, y1_ref, r1_ref):
    x = x_ref[...]
    dn = (((1,), (1,)), ((), ()))
    y1_ref[...] = lax.dot_general(x, wl_ref[...], dn,
                                  preferred_element_type=jnp.float32)
    r1_ref[...] = lax.dot_general(x, wr_ref[...], dn,
                                  preferred_element_type=jnp.float32)


def _agg1_body(y1_hbm, src_hbm, dst_hbm, z64_hbm, z1_hbm, one_hbm,
               p_hbm, cnt_hbm,
               agg_sh, cnt_sh, src_v, dst_v, rows_v, ones_v, stage_v,
               cstage_v, sem):
    c = lax.axis_index("c")
    s = lax.axis_index("s")
    wid = c * NS + s

    pltpu.sync_copy(z64_hbm, stage_v)
    pltpu.sync_copy(z1_hbm, cstage_v)
    pltpu.sync_copy(stage_v, agg_sh.at[pl.ds(s * RP, RP)])
    pltpu.sync_copy(cstage_v, cnt_sh.at[pl.ds(s * RP, RP)])
    pltpu.sync_copy(one_hbm, ones_v)
    pltpu.sync_copy(src_hbm.at[wid], src_v)
    pltpu.sync_copy(dst_hbm.at[wid], dst_v)
    plsc.subcore_barrier()

    def step(g, carry):
        pltpu.async_copy(y1_hbm.at[src_v.at[g]], rows_v, sem).wait()
        pltpu.sync_copy(rows_v, agg_sh.at[dst_v.at[g]], add=True)
        pltpu.sync_copy(ones_v, cnt_sh.at[dst_v.at[g]], add=True)
        return carry

    lax.fori_loop(0, NCHUNK, step, 0)
    plsc.subcore_barrier()

    pltpu.sync_copy(agg_sh.at[pl.ds(s * RP, RP)], stage_v)
    pltpu.sync_copy(cnt_sh.at[pl.ds(s * RP, RP)], cstage_v)
    pltpu.sync_copy(stage_v, p_hbm.at[c, pl.ds(s * RP, RP)])
    pltpu.sync_copy(cstage_v, cnt_hbm.at[c, pl.ds(s * RP, RP)])


def _mid_body(p_ref, cnt_ref, r1_ref, b1_ref, w2l_ref, w2r_ref,
              y2_ref, r2_ref):
    cnt = cnt_ref[0, :N] + cnt_ref[1, :N]
    rcp = 1.0 / jnp.maximum(cnt, 1.0)
    agg = p_ref[0, :N] + p_ref[1, :N]
    h = jax.nn.relu(agg * rcp[:, None] + r1_ref[...] + b1_ref[...][None, :])
    pad = jnp.zeros((NP - N,), jnp.float32)
    y2_ref[...] = jnp.concatenate(
        [jnp.sum(h * w2l_ref[...][0][None, :], axis=1), pad])
    r2_ref[...] = jnp.sum(h * w2r_ref[...][0][None, :], axis=1)


def _agg2_body(y2_hbm, src_hbm, dst_hbm, z1_hbm,
               q_hbm,
               q_sh, src_v, dst_v, vals_v, cstage_v, sem):
    c = lax.axis_index("c")
    s = lax.axis_index("s")
    wid = c * NS + s

    pltpu.sync_copy(z1_hbm, cstage_v)
    pltpu.sync_copy(cstage_v, q_sh.at[pl.ds(s * RP, RP)])
    pltpu.sync_copy(src_hbm.at[wid], src_v)
    pltpu.sync_copy(dst_hbm.at[wid], dst_v)
    plsc.subcore_barrier()

    def step(g, carry):
        pltpu.async_copy(y2_hbm.at[src_v.at[g]], vals_v, sem).wait()
        pltpu.sync_copy(vals_v, q_sh.at[dst_v.at[g]], add=True)
        return carry

    lax.fori_loop(0, NCHUNK, step, 0)
    plsc.subcore_barrier()

    pltpu.sync_copy(q_sh.at[pl.ds(s * RP, RP)], cstage_v)
    pltpu.sync_copy(cstage_v, q_hbm.at[c, pl.ds(s * RP, RP)])


def _out_body(q_ref, cnt_ref, r2_ref, b2_ref, out_ref):
    cnt = cnt_ref[0, :N] + cnt_ref[1, :N]
    rcp = 1.0 / jnp.maximum(cnt, 1.0)
    z = (q_ref[0, :N] + q_ref[1, :N]) * rcp + r2_ref[...] + b2_ref[0]
    out_ref[...] = jax.nn.sigmoid(z)[:, None]


@jax.jit
def kernel(x, edge_index, W1l, W1r, b1, W2l, W2r, b2):
    f32 = jnp.float32
    src = edge_index[0].reshape(NW, NCHUNK, CH)
    dst = edge_index[1].reshape(NW, NCHUNK, CH)
    z64 = jnp.zeros((RP, D_HID), f32)
    z1 = jnp.zeros((RP,), f32)
    ones = jnp.ones((CH,), f32)

    y1, r1 = pl.pallas_call(
        _proj1_body,
        out_shape=[jax.ShapeDtypeStruct((N, D_HID), f32),
                   jax.ShapeDtypeStruct((N, D_HID), f32)],
    )(x, W1l, W1r)

    agg1_partial, cnt_partial = pl.kernel(
        _agg1_body,
        out_type=[jax.ShapeDtypeStruct((NC, NP, D_HID), f32),
                  jax.ShapeDtypeStruct((NC, NP), f32)],
        mesh=_mesh,
        compiler_params=pltpu.CompilerParams(use_tc_tiling_on_sc=False),
        scratch_types=[
            pltpu.VMEM_SHARED((NP, D_HID), f32),
            pltpu.VMEM_SHARED((NP,), f32),
            pltpu.VMEM((NCHUNK, CH), jnp.int32),
            pltpu.VMEM((NCHUNK, CH), jnp.int32),
            pltpu.VMEM((CH, D_HID), f32),
            pltpu.VMEM((CH,), f32),
            pltpu.VMEM((RP, D_HID), f32),
            pltpu.VMEM((RP,), f32),
            pltpu.SemaphoreType.DMA,
        ],
    )(y1, src, dst, z64, z1, ones)

    y2, r2 = pl.pallas_call(
        _mid_body,
        out_shape=[jax.ShapeDtypeStruct((NP,), f32),
                   jax.ShapeDtypeStruct((N,), f32)],
    )(agg1_partial, cnt_partial, r1, b1, W2l, W2r)

    q_partial = pl.kernel(
        _agg2_body,
        out_type=jax.ShapeDtypeStruct((NC, NP), f32),
        mesh=_mesh,
        compiler_params=pltpu.CompilerParams(use_tc_tiling_on_sc=False),
        scratch_types=[
            pltpu.VMEM_SHARED((NP,), f32),
            pltpu.VMEM((NCHUNK, CH), jnp.int32),
            pltpu.VMEM((NCHUNK, CH), jnp.int32),
            pltpu.VMEM((CH,), f32),
            pltpu.VMEM((RP,), f32),
            pltpu.SemaphoreType.DMA,
        ],
    )(y2, src, dst, z1)

    out = pl.pallas_call(
        _out_body,
        out_shape=jax.ShapeDtypeStruct((N, 1), f32),
    )(q_partial, cnt_partial, r2, b2)

    return out


# speedup vs baseline: 16.1625x; 1.5321x over previous
import jax
import jax.numpy as jnp
from jax import lax
from jax.experimental import pallas as pl
from jax.experimental.pallas import tpu as pltpu
from jax.experimental.pallas import tpu_sc as plsc

N = 10000
E = 320000
D_IN = 128
D_HID = 64

NC, NS = 2, 16
NW = NC * NS
EPW = E // NW
CH = 80
NCHUNK = EPW // CH
NP = 10240
RP = NP // NS

_mesh = plsc.VectorSubcoreMesh(core_axis_name="c", subcore_axis_name="s")


def _proj1_body(x_ref, wl_ref, wr_ref, y1_ref, r1_ref):
    x = x_ref[...]
    dn = (((1,), (1,)), ((), ()))
    y1_ref[...] = lax.dot_general(x, wl_ref[...], dn,
                                  preferred_element_type=jnp.float32)
    r1_ref[...] = lax.dot_general(x, wr_ref[...], dn,
                                  preferred_element_type=jnp.float32)


def _agg1_body(y1_hbm, src_hbm, dst_hbm, z64_hbm, z1_hbm, one_hbm,
               p_hbm, cnt_hbm,
               agg_sh, cnt_sh, src_v, dst_v, rows0_v, rows1_v, ones_v,
               stage_v, cstage_v, gsem0, gsem1, ssem0, ssem1, csem):
    c = lax.axis_index("c")
    s = lax.axis_index("s")
    wid = c * NS + s

    pltpu.sync_copy(z64_hbm, stage_v)
    pltpu.sync_copy(z1_hbm, cstage_v)
    pltpu.sync_copy(stage_v, agg_sh.at[pl.ds(s * RP, RP)])
    pltpu.sync_copy(cstage_v, cnt_sh.at[pl.ds(s * RP, RP)])
    pltpu.sync_copy(one_hbm, ones_v)
    pltpu.sync_copy(src_hbm.at[wid], src_v)
    pltpu.sync_copy(dst_hbm.at[wid], dst_v)
    pltpu.async_copy(y1_hbm.at[src_v.at[0]], rows0_v, gsem0)
    plsc.subcore_barrier()

    def step(i, carry):
        a = 2 * i + 1
        pltpu.async_copy(y1_hbm.at[src_v.at[a]], rows1_v, gsem1)
        pltpu.make_async_copy(y1_hbm.at[src_v.at[0]], rows0_v, gsem0).wait()
        pltpu.async_copy(rows0_v, agg_sh.at[dst_v.at[a - 1]], ssem0, add=True)
        pltpu.async_copy(ones_v, cnt_sh.at[dst_v.at[a - 1]], csem, add=True)
        pltpu.make_async_copy(rows0_v, agg_sh.at[dst_v.at[0]], ssem0).wait()
        pltpu.async_copy(y1_hbm.at[src_v.at[a + 1]], rows0_v, gsem0)
        pltpu.make_async_copy(y1_hbm.at[src_v.at[0]], rows1_v, gsem1).wait()
        pltpu.async_copy(rows1_v, agg_sh.at[dst_v.at[a]], ssem1, add=True)
        pltpu.async_copy(ones_v, cnt_sh.at[dst_v.at[a]], csem, add=True)
        pltpu.make_async_copy(rows1_v, agg_sh.at[dst_v.at[0]], ssem1).wait()
        pltpu.make_async_copy(ones_v, cnt_sh.at[dst_v.at[0]], csem).wait()
        pltpu.make_async_copy(ones_v, cnt_sh.at[dst_v.at[0]], csem).wait()
        return carry

    lax.fori_loop(0, (NCHUNK - 1) // 2, step, 0)
    pltpu.make_async_copy(y1_hbm.at[src_v.at[0]], rows0_v, gsem0).wait()
    pltpu.sync_copy(rows0_v, agg_sh.at[dst_v.at[NCHUNK - 1]], add=True)
    pltpu.sync_copy(ones_v, cnt_sh.at[dst_v.at[NCHUNK - 1]], add=True)
    plsc.subcore_barrier()

    pltpu.sync_copy(agg_sh.at[pl.ds(s * RP, RP)], stage_v)
    pltpu.sync_copy(cnt_sh.at[pl.ds(s * RP, RP)], cstage_v)
    pltpu.sync_copy(stage_v, p_hbm.at[c, pl.ds(s * RP, RP)])
    pltpu.sync_copy(cstage_v, cnt_hbm.at[c, pl.ds(s * RP, RP)])


def _mid_body(p_ref, cnt_ref, r1_ref, b1_ref, w2l_ref, w2r_ref,
              y2_ref, r2_ref):
    cnt = cnt_ref[0, :N] + cnt_ref[1, :N]
    rcp = 1.0 / jnp.maximum(cnt, 1.0)
    agg = p_ref[0, :N] + p_ref[1, :N]
    h = jax.nn.relu(agg * rcp[:, None] + r1_ref[...] + b1_ref[...][None, :])
    pad = jnp.zeros((NP - N,), jnp.float32)
    y2_ref[...] = jnp.concatenate(
        [jnp.sum(h * w2l_ref[...][0][None, :], axis=1), pad])
    r2_ref[...] = jnp.sum(h * w2r_ref[...][0][None, :], axis=1)


def _agg2_body(y2_hbm, src_hbm, dst_hbm, z1_hbm,
               q_hbm,
               q_sh, src_v, dst_v, vals0_v, vals1_v, cstage_v,
               gsem0, gsem1, ssem0, ssem1):
    c = lax.axis_index("c")
    s = lax.axis_index("s")
    wid = c * NS + s

    pltpu.sync_copy(z1_hbm, cstage_v)
    pltpu.sync_copy(cstage_v, q_sh.at[pl.ds(s * RP, RP)])
    pltpu.sync_copy(src_hbm.at[wid], src_v)
    pltpu.sync_copy(dst_hbm.at[wid], dst_v)
    pltpu.async_copy(y2_hbm.at[src_v.at[0]], vals0_v, gsem0)
    plsc.subcore_barrier()

    def step(i, carry):
        a = 2 * i + 1
        pltpu.async_copy(y2_hbm.at[src_v.at[a]], vals1_v, gsem1)
        pltpu.make_async_copy(y2_hbm.at[src_v.at[0]], vals0_v, gsem0).wait()
        pltpu.async_copy(vals0_v, q_sh.at[dst_v.at[a - 1]], ssem0, add=True)
        pltpu.make_async_copy(vals0_v, q_sh.at[dst_v.at[0]], ssem0).wait()
        pltpu.async_copy(y2_hbm.at[src_v.at[a + 1]], vals0_v, gsem0)
        pltpu.make_async_copy(y2_hbm.at[src_v.at[0]], vals1_v, gsem1).wait()
        pltpu.async_copy(vals1_v, q_sh.at[dst_v.at[a]], ssem1, add=True)
        pltpu.make_async_copy(vals1_v, q_sh.at[dst_v.at[0]], ssem1).wait()
        return carry

    lax.fori_loop(0, (NCHUNK - 1) // 2, step, 0)
    pltpu.make_async_copy(y2_hbm.at[src_v.at[0]], vals0_v, gsem0).wait()
    pltpu.sync_copy(vals0_v, q_sh.at[dst_v.at[NCHUNK - 1]], add=True)
    plsc.subcore_barrier()

    pltpu.sync_copy(q_sh.at[pl.ds(s * RP, RP)], cstage_v)
    pltpu.sync_copy(cstage_v, q_hbm.at[c, pl.ds(s * RP, RP)])


def _out_body(q_ref, cnt_ref, r2_ref, b2_ref, out_ref):
    cnt = cnt_ref[0, :N] + cnt_ref[1, :N]
    rcp = 1.0 / jnp.maximum(cnt, 1.0)
    z = (q_ref[0, :N] + q_ref[1, :N]) * rcp + r2_ref[...] + b2_ref[0]
    out_ref[...] = jax.nn.sigmoid(z)[:, None]


@jax.jit
def kernel(x, edge_index, W1l, W1r, b1, W2l, W2r, b2):
    f32 = jnp.float32
    src = edge_index[0].reshape(NW, NCHUNK, CH)
    dst = edge_index[1].reshape(NW, NCHUNK, CH)
    z64 = jnp.zeros((RP, D_HID), f32)
    z1 = jnp.zeros((RP,), f32)
    ones = jnp.ones((CH,), f32)

    y1, r1 = pl.pallas_call(
        _proj1_body,
        out_shape=[jax.ShapeDtypeStruct((N, D_HID), f32),
                   jax.ShapeDtypeStruct((N, D_HID), f32)],
    )(x, W1l, W1r)

    agg1_partial, cnt_partial = pl.kernel(
        _agg1_body,
        out_type=[jax.ShapeDtypeStruct((NC, NP, D_HID), f32),
                  jax.ShapeDtypeStruct((NC, NP), f32)],
        mesh=_mesh,
        compiler_params=pltpu.CompilerParams(use_tc_tiling_on_sc=False),
        scratch_types=[
            pltpu.VMEM_SHARED((NP, D_HID), f32),
            pltpu.VMEM_SHARED((NP,), f32),
            pltpu.VMEM((NCHUNK, CH), jnp.int32),
            pltpu.VMEM((NCHUNK, CH), jnp.int32),
            pltpu.VMEM((CH, D_HID), f32),
            pltpu.VMEM((CH, D_HID), f32),
            pltpu.VMEM((CH,), f32),
            pltpu.VMEM((RP, D_HID), f32),
            pltpu.VMEM((RP,), f32),
            pltpu.SemaphoreType.DMA,
            pltpu.SemaphoreType.DMA,
            pltpu.SemaphoreType.DMA,
            pltpu.SemaphoreType.DMA,
            pltpu.SemaphoreType.DMA,
        ],
    )(y1, src, dst, z64, z1, ones)

    y2, r2 = pl.pallas_call(
        _mid_body,
        out_shape=[jax.ShapeDtypeStruct((NP,), f32),
                   jax.ShapeDtypeStruct((N,), f32)],
    )(agg1_partial, cnt_partial, r1, b1, W2l, W2r)

    q_partial = pl.kernel(
        _agg2_body,
        out_type=jax.ShapeDtypeStruct((NC, NP), f32),
        mesh=_mesh,
        compiler_params=pltpu.CompilerParams(use_tc_tiling_on_sc=False),
        scratch_types=[
            pltpu.VMEM_SHARED((NP,), f32),
            pltpu.VMEM((NCHUNK, CH), jnp.int32),
            pltpu.VMEM((NCHUNK, CH), jnp.int32),
            pltpu.VMEM((CH,), f32),
            pltpu.VMEM((CH,), f32),
            pltpu.VMEM((RP,), f32),
            pltpu.SemaphoreType.DMA,
            pltpu.SemaphoreType.DMA,
            pltpu.SemaphoreType.DMA,
            pltpu.SemaphoreType.DMA,
        ],
    )(y2, src, dst, z1)

    out = pl.pallas_call(
        _out_body,
        out_shape=jax.ShapeDtypeStruct((N, 1), f32),
    )(q_partial, cnt_partial, r2, b2)

    return out
